# baseline (speedup 1.0000x reference)
import functools

import jax
import jax.numpy as jnp
from jax import lax
from jax.experimental import pallas as pl
from jax.experimental.pallas import tpu as pltpu
from jax.experimental.pallas import tpu_sc as plsc

_SMALL = 1e-8
_BN_EPS = 1e-5
_D = 64
_NC = 2
_NS = 16
_NW = _NC * _NS
_L16 = 16


def _make_pool(B, Lp):
    nch = Lp // _L16
    spw = B // _NW
    chunks = []
    off = 0
    while off < Lp:
        cl = min(128, Lp - off)
        chunks.append((off, cl))
        off += cl

    mesh = plsc.VectorSubcoreMesh(core_axis_name="c", subcore_axis_name="s")

    @functools.partial(
        pl.kernel,
        out_type=jax.ShapeDtypeStruct((B, _D), jnp.float32),
        mesh=mesh,
        compiler_params=pltpu.CompilerParams(
            needs_layout_passes=False, use_tc_tiling_on_sc=False),
        scratch_types=[
            pltpu.VMEM((spw + 1, Lp), jnp.int32),
            pltpu.VMEM((2, Lp, _D), jnp.float32),
            pltpu.VMEM((2, Lp), jnp.float32),
            pltpu.VMEM((Lp,), jnp.float32),
            pltpu.VMEM((Lp,), jnp.float32),
            pltpu.VMEM((spw, _D), jnp.float32),
            pltpu.SemaphoreType.DMA,
            pltpu.SemaphoreType.DMA,
        ],
    )
    def pool(seqs, table, t, scale, bias, out,
             idx_v, rows_v, dots_v, scale_v, bias_v, out_v, sem0, sem1):
        wid = lax.axis_index("s") * _NC + lax.axis_index("c")
        base = wid * spw
        pltpu.sync_copy(scale, scale_v)
        pltpu.sync_copy(bias, bias_v)
        pltpu.sync_copy(seqs.at[pl.ds(base, spw)], idx_v.at[pl.ds(0, spw)])
        for c in range(nch):
            idx_v[spw, pl.ds(c * _L16, _L16)] = jnp.zeros((_L16,), jnp.int32)
        sems = [sem0, sem1]

        def start(s, b):
            for (o, c) in chunks:
                pltpu.async_copy(table.at[idx_v.at[s, pl.ds(o, c)]],
                                 rows_v.at[b, pl.ds(o, c)], sems[b])
                pltpu.async_copy(t.at[idx_v.at[s, pl.ds(o, c)]],
                                 dots_v.at[b, pl.ds(o, c)], sems[b])

        def wait(b):
            for (o, c) in chunks:
                pltpu.make_async_copy(table.at[idx_v.at[0, pl.ds(o, c)]],
                                      rows_v.at[b, pl.ds(o, c)],
                                      sems[b]).wait()
                pltpu.make_async_copy(t.at[idx_v.at[0, pl.ds(o, c)]],
                                      dots_v.at[b, pl.ds(o, c)],
                                      sems[b]).wait()

        def compute(s, b):
            rb = rows_v.at[b]

            def chunk_body(c, carry):
                a0, a1, a2, a3, wsv = carry
                acc = dots_v[b, pl.ds(c * _L16, _L16)]
                ids = idx_v[s, pl.ds(c * _L16, _L16)]
                x = acc * scale_v[pl.ds(c * _L16, _L16)] \
                    + bias_v[pl.ds(c * _L16, _L16)]
                sig = 1.0 / (1.0 + jnp.exp(-x))
                wv16 = jnp.where(ids != 0, sig, 0.0)
                accs = [a0, a1, a2, a3]
                for k in range(_L16):
                    l = c * _L16 + k
                    ws = wv16[k]
                    for q in range(4):
                        accs[q] = accs[q] + ws * rb[l, pl.ds(q * 16, 16)]
                return (accs[0], accs[1], accs[2], accs[3], wsv + wv16)

            z = jnp.zeros((_L16,), jnp.float32)
            a0, a1, a2, a3, wsv = lax.fori_loop(
                0, nch, chunk_body, (z, z, z, z, z))
            r = 1.0 / (jnp.zeros((_L16,), jnp.float32) + jnp.sum(wsv) + _SMALL)
            out_v[s, pl.ds(0, 16)] = a0 * r
            out_v[s, pl.ds(16, 16)] = a1 * r
            out_v[s, pl.ds(32, 16)] = a2 * r
            out_v[s, pl.ds(48, 16)] = a3 * r

        start(0, 0)

        def pair_body(g, _):
            s0 = 2 * g
            start(s0 + 1, 1)
            wait(0)
            compute(s0, 0)
            start(s0 + 2, 0)
            wait(1)
            compute(s0 + 1, 1)
            return 0

        lax.fori_loop(0, spw // 2, pair_body, 0)
        wait(0)
        pltpu.sync_copy(out_v, out.at[pl.ds(base, spw)])

    return pool


def _dot_body(w_ref, t_ref, o_ref):
    o_ref[...] = lax.dot_general(
        w_ref[...], t_ref[...], (((1,), (1,)), ((), ())),
        preferred_element_type=jnp.float32)


def _dot_table(table, W):
    V = table.shape[0]
    BK = 8192
    NB = (V + BK - 1) // BK
    t2d = pl.pallas_call(
        _dot_body,
        grid=(NB,),
        in_specs=[
            pl.BlockSpec((1, _D), lambda i: (0, 0)),
            pl.BlockSpec((BK, _D), lambda i: (i, 0)),
        ],
        out_specs=pl.BlockSpec((1, BK), lambda i: (0, i)),
        out_shape=jax.ShapeDtypeStruct((1, NB * BK), jnp.float32),
    )(W.reshape(1, _D), table)
    return t2d.reshape(-1)[:V]


def _sim_body(q_ref, c_ref, o_ref):
    q = q_ref[...]
    c = c_ref[...]
    qn = q / (jnp.sqrt(jnp.sum(q * q, axis=1, keepdims=True)) + _SMALL)
    cn = c / (jnp.sqrt(jnp.sum(c * c, axis=1, keepdims=True)) + _SMALL)
    o_ref[...] = lax.dot_general(
        qn, cn, (((1,), (1,)), ((), ())),
        preferred_element_type=jnp.float32)


def _similarity(qm, cm):
    B = qm.shape[0]
    BQ, BC = 512, 1024
    return pl.pallas_call(
        _sim_body,
        grid=(B // BQ, B // BC),
        in_specs=[
            pl.BlockSpec((BQ, _D), lambda i, j: (i, 0)),
            pl.BlockSpec((BC, _D), lambda i, j: (j, 0)),
        ],
        out_specs=pl.BlockSpec((BQ, BC), lambda i, j: (i, j)),
        out_shape=jax.ShapeDtypeStruct((B, B), jnp.float32),
    )(qm, cm)


def _prep(seqs, gamma, beta, mean, var, L):
    Lp = ((L + _L16 - 1) // _L16) * _L16
    seqs_p = jnp.pad(seqs.astype(jnp.int32), ((0, 0), (0, Lp - L)))
    sc = gamma / jnp.sqrt(var + _BN_EPS)
    bs = beta - mean * sc
    sc_p = jnp.pad(sc, (0, Lp - L), constant_values=1.0)
    bs_p = jnp.pad(bs, (0, Lp - L))
    return seqs_p, sc_p, bs_p, Lp


def kernel(code_seqs, query_seqs, code_table, query_table, Wc, Wq,
           gamma_c, beta_c, mean_c, var_c, gamma_q, beta_q, mean_q, var_q):
    B, LC = code_seqs.shape
    _, LQ = query_seqs.shape
    cs, csc, cbs, LCp = _prep(code_seqs, gamma_c, beta_c, mean_c, var_c, LC)
    qs, qsc, qbs, LQp = _prep(query_seqs, gamma_q, beta_q, mean_q, var_q, LQ)
    tc = _dot_table(code_table, Wc)
    tq = _dot_table(query_table, Wq)
    cmean = _make_pool(B, LCp)(cs, code_table, tc, csc, cbs)
    qmean = _make_pool(B, LQp)(qs, query_table, tq, qsc, qbs)
    return _similarity(qmean, cmean)


# speedup vs baseline: 1.0504x; 1.0504x over previous
import functools

import jax
import jax.numpy as jnp
from jax import lax
from jax.experimental import pallas as pl
from jax.experimental.pallas import tpu as pltpu
from jax.experimental.pallas import tpu_sc as plsc

_SMALL = 1e-8
_BN_EPS = 1e-5
_D = 64
_NC = 2
_NS = 16
_NW = _NC * _NS
_L16 = 16


def _make_pool(B, Lp):
    nch = Lp // _L16
    spw = B // _NW
    chunks = []
    off = 0
    while off < Lp:
        cl = min(128, Lp - off)
        chunks.append((off, cl))
        off += cl

    mesh = plsc.VectorSubcoreMesh(core_axis_name="c", subcore_axis_name="s")

    nbuf = 4
    scratch_types = [
            pltpu.VMEM((spw + nbuf - 1, Lp), jnp.int32),
            pltpu.VMEM((nbuf, Lp, _D), jnp.float32),
            pltpu.VMEM((Lp,), jnp.float32),
            pltpu.VMEM((Lp,), jnp.float32),
            pltpu.VMEM((_D,), jnp.float32),
            pltpu.VMEM((spw, _D), jnp.float32),
        ] + [pltpu.SemaphoreType.DMA] * nbuf

    @functools.partial(
        pl.kernel,
        out_type=jax.ShapeDtypeStruct((B, _D), jnp.float32),
        mesh=mesh,
        compiler_params=pltpu.CompilerParams(
            needs_layout_passes=False, use_tc_tiling_on_sc=False),
        scratch_types=scratch_types,
    )
    def pool(seqs, table, scale, bias, w, out,
             idx_v, rows_v, scale_v, bias_v, w_v, out_v, *sems):
        wid = lax.axis_index("s") * _NC + lax.axis_index("c")
        base = wid * spw
        pltpu.sync_copy(scale, scale_v)
        pltpu.sync_copy(bias, bias_v)
        pltpu.sync_copy(w, w_v)
        pltpu.sync_copy(seqs.at[pl.ds(base, spw)], idx_v.at[pl.ds(0, spw)])
        for j in range(nbuf - 1):
            for c in range(nch):
                idx_v[spw + j, pl.ds(c * _L16, _L16)] = \
                    jnp.zeros((_L16,), jnp.int32)
        wregs = [w_v[pl.ds(q * 16, 16)] for q in range(4)]

        def start(s, b):
            for (o, c) in chunks:
                pltpu.async_copy(table.at[idx_v.at[s, pl.ds(o, c)]],
                                 rows_v.at[b, pl.ds(o, c)], sems[b])

        def wait(b):
            for (o, c) in chunks:
                pltpu.make_async_copy(table.at[idx_v.at[0, pl.ds(o, c)]],
                                      rows_v.at[b, pl.ds(o, c)],
                                      sems[b]).wait()

        def compute(s, b):
            rb = rows_v.at[b]
            lanes = lax.iota(jnp.int32, _L16)

            def chunk_body(c, carry):
                a0, a1, a2, a3, wsv = carry
                terms = []
                for k in range(_L16):
                    l = c * _L16 + k
                    m = rb[l, pl.ds(0, 16)] * wregs[0]
                    for q in range(1, 4):
                        m = m + rb[l, pl.ds(q * 16, 16)] * wregs[q]
                    terms.append(
                        jnp.where(lanes == k, jnp.sum(m), 0.0))
                while len(terms) > 1:
                    terms = [a + b2 for a, b2 in zip(terms[::2], terms[1::2])]
                dots = terms[0]
                ids = idx_v[s, pl.ds(c * _L16, _L16)]
                x = dots * scale_v[pl.ds(c * _L16, _L16)] \
                    + bias_v[pl.ds(c * _L16, _L16)]
                sig = 1.0 / (1.0 + jnp.exp(-x))
                wv16 = jnp.where(ids != 0, sig, 0.0)
                accs = [a0, a1, a2, a3]
                for k in range(_L16):
                    l = c * _L16 + k
                    ws = wv16[k]
                    for q in range(4):
                        accs[q] = accs[q] + ws * rb[l, pl.ds(q * 16, 16)]
                return (accs[0], accs[1], accs[2], accs[3], wsv + wv16)

            z = jnp.zeros((_L16,), jnp.float32)
            a0, a1, a2, a3, wsv = lax.fori_loop(
                0, nch, chunk_body, (z, z, z, z, z))
            r = 1.0 / (jnp.zeros((_L16,), jnp.float32) + jnp.sum(wsv) + _SMALL)
            out_v[s, pl.ds(0, 16)] = a0 * r
            out_v[s, pl.ds(16, 16)] = a1 * r
            out_v[s, pl.ds(32, 16)] = a2 * r
            out_v[s, pl.ds(48, 16)] = a3 * r

        for j in range(nbuf - 1):
            start(j, j)

        def group_body(g, _):
            for b in range(nbuf):
                s = nbuf * g + b
                start(s + nbuf - 1, (b + nbuf - 1) % nbuf)
                wait(b)
                compute(s, b)
            return 0

        lax.fori_loop(0, spw // nbuf, group_body, 0)
        for j in range(nbuf - 1):
            wait(j)
        pltpu.sync_copy(out_v, out.at[pl.ds(base, spw)])

    return pool


def _dot_body(w_ref, t_ref, o_ref):
    o_ref[...] = lax.dot_general(
        w_ref[...], t_ref[...], (((1,), (1,)), ((), ())),
        preferred_element_type=jnp.float32)


def _dot_table(table, W):
    V = table.shape[0]
    BK = 8192
    NB = (V + BK - 1) // BK
    t2d = pl.pallas_call(
        _dot_body,
        grid=(NB,),
        in_specs=[
            pl.BlockSpec((1, _D), lambda i: (0, 0)),
            pl.BlockSpec((BK, _D), lambda i: (i, 0)),
        ],
        out_specs=pl.BlockSpec((1, BK), lambda i: (0, i)),
        out_shape=jax.ShapeDtypeStruct((1, NB * BK), jnp.float32),
    )(W.reshape(1, _D), table)
    return t2d.reshape(-1)[:V]


def _sim_body(q_ref, c_ref, o_ref):
    q = q_ref[...]
    c = c_ref[...]
    qn = q / (jnp.sqrt(jnp.sum(q * q, axis=1, keepdims=True)) + _SMALL)
    cn = c / (jnp.sqrt(jnp.sum(c * c, axis=1, keepdims=True)) + _SMALL)
    o_ref[...] = lax.dot_general(
        qn, cn, (((1,), (1,)), ((), ())),
        preferred_element_type=jnp.float32)


def _similarity(qm, cm):
    B = qm.shape[0]
    BQ, BC = 512, 1024
    return pl.pallas_call(
        _sim_body,
        grid=(B // BQ, B // BC),
        in_specs=[
            pl.BlockSpec((BQ, _D), lambda i, j: (i, 0)),
            pl.BlockSpec((BC, _D), lambda i, j: (j, 0)),
        ],
        out_specs=pl.BlockSpec((BQ, BC), lambda i, j: (i, j)),
        out_shape=jax.ShapeDtypeStruct((B, B), jnp.float32),
    )(qm, cm)


def _prep(seqs, gamma, beta, mean, var, L):
    Lp = ((L + _L16 - 1) // _L16) * _L16
    seqs_p = jnp.pad(seqs.astype(jnp.int32), ((0, 0), (0, Lp - L)))
    sc = gamma / jnp.sqrt(var + _BN_EPS)
    bs = beta - mean * sc
    sc_p = jnp.pad(sc, (0, Lp - L), constant_values=1.0)
    bs_p = jnp.pad(bs, (0, Lp - L))
    return seqs_p, sc_p, bs_p, Lp


def kernel(code_seqs, query_seqs, code_table, query_table, Wc, Wq,
           gamma_c, beta_c, mean_c, var_c, gamma_q, beta_q, mean_q, var_q):
    B, LC = code_seqs.shape
    _, LQ = query_seqs.shape
    cs, csc, cbs, LCp = _prep(code_seqs, gamma_c, beta_c, mean_c, var_c, LC)
    qs, qsc, qbs, LQp = _prep(query_seqs, gamma_q, beta_q, mean_q, var_q, LQ)
    cmean = _make_pool(B, LCp)(cs, code_table, csc, cbs, Wc.reshape(-1))
    qmean = _make_pool(B, LQp)(qs, query_table, qsc, qbs, Wq.reshape(-1))
    return _similarity(qmean, cmean)


# speedup vs baseline: 1.0553x; 1.0046x over previous
import functools

import jax
import jax.numpy as jnp
from jax import lax
from jax.experimental import pallas as pl
from jax.experimental.pallas import tpu as pltpu
from jax.experimental.pallas import tpu_sc as plsc

_SMALL = 1e-8
_BN_EPS = 1e-5
_D = 64
_NC = 2
_NS = 16
_NW = _NC * _NS
_L16 = 16


def _make_pool(B, Lp):
    nch = Lp // _L16
    spw = B // _NW
    chunks = []
    off = 0
    while off < Lp:
        cl = min(128, Lp - off)
        chunks.append((off, cl))
        off += cl

    mesh = plsc.VectorSubcoreMesh(core_axis_name="c", subcore_axis_name="s")

    nbuf = 4
    scratch_types = [
            pltpu.VMEM((spw + nbuf - 1, Lp), jnp.int32),
            pltpu.VMEM((nbuf, Lp, _D), jnp.float32),
            pltpu.VMEM((Lp,), jnp.float32),
            pltpu.VMEM((Lp,), jnp.float32),
            pltpu.VMEM((_D,), jnp.float32),
            pltpu.VMEM((spw, _D), jnp.float32),
        ] + [pltpu.SemaphoreType.DMA] * nbuf

    @functools.partial(
        pl.kernel,
        out_type=jax.ShapeDtypeStruct((B, _D), jnp.float32),
        mesh=mesh,
        compiler_params=pltpu.CompilerParams(
            needs_layout_passes=False, use_tc_tiling_on_sc=False),
        scratch_types=scratch_types,
    )
    def pool(seqs, table, scale, bias, w, out,
             idx_v, rows_v, scale_v, bias_v, w_v, out_v, *sems):
        wid = lax.axis_index("s") * _NC + lax.axis_index("c")
        base = wid * spw
        pltpu.sync_copy(scale, scale_v)
        pltpu.sync_copy(bias, bias_v)
        pltpu.sync_copy(w, w_v)
        pltpu.sync_copy(seqs.at[pl.ds(base, spw)], idx_v.at[pl.ds(0, spw)])
        for j in range(nbuf - 1):
            for c in range(nch):
                idx_v[spw + j, pl.ds(c * _L16, _L16)] = \
                    jnp.zeros((_L16,), jnp.int32)
        wregs = [w_v[pl.ds(q * 16, 16)] for q in range(4)]

        def start(s, b):
            for (o, c) in chunks:
                pltpu.async_copy(table.at[idx_v.at[s, pl.ds(o, c)]],
                                 rows_v.at[b, pl.ds(o, c)], sems[b])

        def wait(b):
            for (o, c) in chunks:
                pltpu.make_async_copy(table.at[idx_v.at[0, pl.ds(o, c)]],
                                      rows_v.at[b, pl.ds(o, c)],
                                      sems[b]).wait()

        def compute(s, b):
            rb = rows_v.at[b]
            lanes = lax.iota(jnp.int32, _L16)
            if True:
                z16 = rb[0, pl.ds(0, 16)]
                out_v[s, pl.ds(0, 16)] = z16
                out_v[s, pl.ds(16, 16)] = z16
                out_v[s, pl.ds(32, 16)] = z16
                out_v[s, pl.ds(48, 16)] = z16
                return

            def chunk_body(c, carry):
                a0, a1, a2, a3, wsv = carry
                terms = []
                for k in range(_L16):
                    l = c * _L16 + k
                    m = rb[l, pl.ds(0, 16)] * wregs[0]
                    for q in range(1, 4):
                        m = m + rb[l, pl.ds(q * 16, 16)] * wregs[q]
                    terms.append(
                        jnp.where(lanes == k, jnp.sum(m), 0.0))
                while len(terms) > 1:
                    terms = [a + b2 for a, b2 in zip(terms[::2], terms[1::2])]
                dots = terms[0]
                ids = idx_v[s, pl.ds(c * _L16, _L16)]
                x = dots * scale_v[pl.ds(c * _L16, _L16)] \
                    + bias_v[pl.ds(c * _L16, _L16)]
                sig = 1.0 / (1.0 + jnp.exp(-x))
                wv16 = jnp.where(ids != 0, sig, 0.0)
                accs = [a0, a1, a2, a3]
                for k in range(_L16):
                    l = c * _L16 + k
                    ws = wv16[k]
                    for q in range(4):
                        accs[q] = accs[q] + ws * rb[l, pl.ds(q * 16, 16)]
                return (accs[0], accs[1], accs[2], accs[3], wsv + wv16)

            z = jnp.zeros((_L16,), jnp.float32)
            a0, a1, a2, a3, wsv = lax.fori_loop(
                0, nch, chunk_body, (z, z, z, z, z))
            r = 1.0 / (jnp.zeros((_L16,), jnp.float32) + jnp.sum(wsv) + _SMALL)
            out_v[s, pl.ds(0, 16)] = a0 * r
            out_v[s, pl.ds(16, 16)] = a1 * r
            out_v[s, pl.ds(32, 16)] = a2 * r
            out_v[s, pl.ds(48, 16)] = a3 * r

        for j in range(nbuf - 1):
            start(j, j)

        def group_body(g, _):
            for b in range(nbuf):
                s = nbuf * g + b
                start(s + nbuf - 1, (b + nbuf - 1) % nbuf)
                wait(b)
                compute(s, b)
            return 0

        lax.fori_loop(0, spw // nbuf, group_body, 0)
        for j in range(nbuf - 1):
            wait(j)
        pltpu.sync_copy(out_v, out.at[pl.ds(base, spw)])

    return pool


def _dot_body(w_ref, t_ref, o_ref):
    o_ref[...] = lax.dot_general(
        w_ref[...], t_ref[...], (((1,), (1,)), ((), ())),
        preferred_element_type=jnp.float32)


def _dot_table(table, W):
    V = table.shape[0]
    BK = 8192
    NB = (V + BK - 1) // BK
    t2d = pl.pallas_call(
        _dot_body,
        grid=(NB,),
        in_specs=[
            pl.BlockSpec((1, _D), lambda i: (0, 0)),
            pl.BlockSpec((BK, _D), lambda i: (i, 0)),
        ],
        out_specs=pl.BlockSpec((1, BK), lambda i: (0, i)),
        out_shape=jax.ShapeDtypeStruct((1, NB * BK), jnp.float32),
    )(W.reshape(1, _D), table)
    return t2d.reshape(-1)[:V]


def _sim_body(q_ref, c_ref, o_ref):
    q = q_ref[...]
    c = c_ref[...]
    qn = q / (jnp.sqrt(jnp.sum(q * q, axis=1, keepdims=True)) + _SMALL)
    cn = c / (jnp.sqrt(jnp.sum(c * c, axis=1, keepdims=True)) + _SMALL)
    o_ref[...] = lax.dot_general(
        qn, cn, (((1,), (1,)), ((), ())),
        preferred_element_type=jnp.float32)


def _similarity(qm, cm):
    B = qm.shape[0]
    BQ, BC = 512, 1024
    return pl.pallas_call(
        _sim_body,
        grid=(B // BQ, B // BC),
        in_specs=[
            pl.BlockSpec((BQ, _D), lambda i, j: (i, 0)),
            pl.BlockSpec((BC, _D), lambda i, j: (j, 0)),
        ],
        out_specs=pl.BlockSpec((BQ, BC), lambda i, j: (i, j)),
        out_shape=jax.ShapeDtypeStruct((B, B), jnp.float32),
    )(qm, cm)


def _prep(seqs, gamma, beta, mean, var, L):
    Lp = ((L + _L16 - 1) // _L16) * _L16
    seqs_p = jnp.pad(seqs.astype(jnp.int32), ((0, 0), (0, Lp - L)))
    sc = gamma / jnp.sqrt(var + _BN_EPS)
    bs = beta - mean * sc
    sc_p = jnp.pad(sc, (0, Lp - L), constant_values=1.0)
    bs_p = jnp.pad(bs, (0, Lp - L))
    return seqs_p, sc_p, bs_p, Lp


def kernel(code_seqs, query_seqs, code_table, query_table, Wc, Wq,
           gamma_c, beta_c, mean_c, var_c, gamma_q, beta_q, mean_q, var_q):
    B, LC = code_seqs.shape
    _, LQ = query_seqs.shape
    cs, csc, cbs, LCp = _prep(code_seqs, gamma_c, beta_c, mean_c, var_c, LC)
    qs, qsc, qbs, LQp = _prep(query_seqs, gamma_q, beta_q, mean_q, var_q, LQ)
    cmean = _make_pool(B, LCp)(cs, code_table, csc, cbs, Wc.reshape(-1))
    qmean = _make_pool(B, LQp)(qs, query_table, qsc, qbs, Wq.reshape(-1))
    return _similarity(qmean, cmean)


# speedup vs baseline: 2.8431x; 2.6942x over previous
import functools

import jax
import jax.numpy as jnp
from jax import lax
from jax.experimental import pallas as pl
from jax.experimental.pallas import tpu as pltpu
from jax.experimental.pallas import tpu_sc as plsc

_SMALL = 1e-8
_BN_EPS = 1e-5
_D = 64
_NC = 2
_NS = 16
_NW = _NC * _NS
_L16 = 16


def _make_pool(B, Lp):
    nch = Lp // _L16
    spw = B // _NW
    chunks = []
    off = 0
    while off < Lp:
        cl = min(128, Lp - off)
        chunks.append((off, cl))
        off += cl

    mesh = plsc.VectorSubcoreMesh(core_axis_name="c", subcore_axis_name="s")

    nbuf = 4
    scratch_types = [
            pltpu.VMEM((spw + nbuf - 1, Lp), jnp.int32),
            pltpu.VMEM((nbuf, Lp, _D), jnp.float32),
            pltpu.VMEM((Lp,), jnp.float32),
            pltpu.VMEM((Lp,), jnp.float32),
            pltpu.VMEM((_D,), jnp.float32),
            pltpu.VMEM((spw, _D), jnp.float32),
        ] + [pltpu.SemaphoreType.DMA] * nbuf

    @functools.partial(
        pl.kernel,
        out_type=jax.ShapeDtypeStruct((B, _D), jnp.float32),
        mesh=mesh,
        compiler_params=pltpu.CompilerParams(
            needs_layout_passes=False, use_tc_tiling_on_sc=False),
        scratch_types=scratch_types,
    )
    def pool(seqs, table, scale, bias, w, out,
             idx_v, rows_v, scale_v, bias_v, w_v, out_v, *sems):
        wid = lax.axis_index("s") * _NC + lax.axis_index("c")
        base = wid * spw
        pltpu.sync_copy(scale, scale_v)
        pltpu.sync_copy(bias, bias_v)
        pltpu.sync_copy(w, w_v)
        pltpu.sync_copy(seqs.at[pl.ds(base, spw)], idx_v.at[pl.ds(0, spw)])
        for j in range(nbuf - 1):
            for c in range(nch):
                idx_v[spw + j, pl.ds(c * _L16, _L16)] = \
                    wid * 503 + j * 67 + c * _L16 + lax.iota(jnp.int32, _L16)
        wregs = [w_v[pl.ds(q * 16, 16)] for q in range(4)]

        def start(s, b):
            for (o, c) in chunks:
                pltpu.async_copy(table.at[idx_v.at[s, pl.ds(o, c)]],
                                 rows_v.at[b, pl.ds(o, c)], sems[b])

        def wait(b):
            for (o, c) in chunks:
                pltpu.make_async_copy(table.at[idx_v.at[0, pl.ds(o, c)]],
                                      rows_v.at[b, pl.ds(o, c)],
                                      sems[b]).wait()

        def compute(s, b):
            rb = rows_v.at[b]
            lanes = lax.iota(jnp.int32, _L16)

            def chunk_body(c, carry):
                a0, a1, a2, a3, wsv = carry
                terms = []
                for k in range(_L16):
                    l = c * _L16 + k
                    m = rb[l, pl.ds(0, 16)] * wregs[0]
                    for q in range(1, 4):
                        m = m + rb[l, pl.ds(q * 16, 16)] * wregs[q]
                    terms.append(
                        jnp.where(lanes == k, jnp.sum(m), 0.0))
                while len(terms) > 1:
                    terms = [a + b2 for a, b2 in zip(terms[::2], terms[1::2])]
                dots = terms[0]
                ids = idx_v[s, pl.ds(c * _L16, _L16)]
                x = dots * scale_v[pl.ds(c * _L16, _L16)] \
                    + bias_v[pl.ds(c * _L16, _L16)]
                sig = 1.0 / (1.0 + jnp.exp(-x))
                wv16 = jnp.where(ids != 0, sig, 0.0)
                accs = [a0, a1, a2, a3]
                for k in range(_L16):
                    l = c * _L16 + k
                    ws = wv16[k]
                    for q in range(4):
                        accs[q] = accs[q] + ws * rb[l, pl.ds(q * 16, 16)]
                return (accs[0], accs[1], accs[2], accs[3], wsv + wv16)

            z = jnp.zeros((_L16,), jnp.float32)
            a0, a1, a2, a3, wsv = lax.fori_loop(
                0, nch, chunk_body, (z, z, z, z, z))
            r = 1.0 / (jnp.zeros((_L16,), jnp.float32) + jnp.sum(wsv) + _SMALL)
            out_v[s, pl.ds(0, 16)] = a0 * r
            out_v[s, pl.ds(16, 16)] = a1 * r
            out_v[s, pl.ds(32, 16)] = a2 * r
            out_v[s, pl.ds(48, 16)] = a3 * r

        for j in range(nbuf - 1):
            start(j, j)

        def group_body(g, _):
            for b in range(nbuf):
                s = nbuf * g + b
                start(s + nbuf - 1, (b + nbuf - 1) % nbuf)
                wait(b)
                compute(s, b)
            return 0

        lax.fori_loop(0, spw // nbuf, group_body, 0)
        for j in range(nbuf - 1):
            wait(j)
        pltpu.sync_copy(out_v, out.at[pl.ds(base, spw)])

    return pool


def _dot_body(w_ref, t_ref, o_ref):
    o_ref[...] = lax.dot_general(
        w_ref[...], t_ref[...], (((1,), (1,)), ((), ())),
        preferred_element_type=jnp.float32)


def _dot_table(table, W):
    V = table.shape[0]
    BK = 8192
    NB = (V + BK - 1) // BK
    t2d = pl.pallas_call(
        _dot_body,
        grid=(NB,),
        in_specs=[
            pl.BlockSpec((1, _D), lambda i: (0, 0)),
            pl.BlockSpec((BK, _D), lambda i: (i, 0)),
        ],
        out_specs=pl.BlockSpec((1, BK), lambda i: (0, i)),
        out_shape=jax.ShapeDtypeStruct((1, NB * BK), jnp.float32),
    )(W.reshape(1, _D), table)
    return t2d.reshape(-1)[:V]


def _sim_body(q_ref, c_ref, o_ref):
    q = q_ref[...]
    c = c_ref[...]
    qn = q / (jnp.sqrt(jnp.sum(q * q, axis=1, keepdims=True)) + _SMALL)
    cn = c / (jnp.sqrt(jnp.sum(c * c, axis=1, keepdims=True)) + _SMALL)
    o_ref[...] = lax.dot_general(
        qn, cn, (((1,), (1,)), ((), ())),
        preferred_element_type=jnp.float32)


def _similarity(qm, cm):
    B = qm.shape[0]
    BQ, BC = 512, 1024
    return pl.pallas_call(
        _sim_body,
        grid=(B // BQ, B // BC),
        in_specs=[
            pl.BlockSpec((BQ, _D), lambda i, j: (i, 0)),
            pl.BlockSpec((BC, _D), lambda i, j: (j, 0)),
        ],
        out_specs=pl.BlockSpec((BQ, BC), lambda i, j: (i, j)),
        out_shape=jax.ShapeDtypeStruct((B, B), jnp.float32),
    )(qm, cm)


def _prep(seqs, gamma, beta, mean, var, L, V):
    Lp = ((L + _L16 - 1) // _L16) * _L16
    B = seqs.shape[0]
    spread = (jax.lax.broadcasted_iota(jnp.int32, (B, Lp), 0) * 131
              + jax.lax.broadcasted_iota(jnp.int32, (B, Lp), 1)) % V
    pos = jax.lax.broadcasted_iota(jnp.int32, (B, Lp), 1)
    seqs_p = jnp.where(pos < L,
                       jnp.pad(seqs.astype(jnp.int32),
                               ((0, 0), (0, Lp - L))),
                       spread)
    sc = gamma / jnp.sqrt(var + _BN_EPS)
    bs = beta - mean * sc
    sc_p = jnp.pad(sc, (0, Lp - L))
    bs_p = jnp.pad(bs, (0, Lp - L), constant_values=-60.0)
    return seqs_p, sc_p, bs_p, Lp


def kernel(code_seqs, query_seqs, code_table, query_table, Wc, Wq,
           gamma_c, beta_c, mean_c, var_c, gamma_q, beta_q, mean_q, var_q):
    B, LC = code_seqs.shape
    _, LQ = query_seqs.shape
    cs, csc, cbs, LCp = _prep(code_seqs, gamma_c, beta_c, mean_c, var_c, LC,
                              code_table.shape[0])
    qs, qsc, qbs, LQp = _prep(query_seqs, gamma_q, beta_q, mean_q, var_q, LQ,
                              query_table.shape[0])
    cmean = _make_pool(B, LCp)(cs, code_table, csc, cbs, Wc.reshape(-1))
    qmean = _make_pool(B, LQp)(qs, query_table, qsc, qbs, Wq.reshape(-1))
    return _similarity(qmean, cmean)


# speedup vs baseline: 3.1585x; 1.1109x over previous
import functools

import jax
import jax.numpy as jnp
from jax import lax
from jax.experimental import pallas as pl
from jax.experimental.pallas import tpu as pltpu
from jax.experimental.pallas import tpu_sc as plsc

_SMALL = 1e-8
_BN_EPS = 1e-5
_D = 64
_NC = 2
_NS = 16
_NW = _NC * _NS
_L16 = 16


def _make_pool(B, Lp, nbuf):
    nch = Lp // _L16
    spw = B // _NW
    chunks = []
    off = 0
    while off < Lp:
        cl = min(128, Lp - off)
        chunks.append((off, cl))
        off += cl

    mesh = plsc.VectorSubcoreMesh(core_axis_name="c", subcore_axis_name="s")

    scratch_types = [
            pltpu.VMEM((spw + nbuf - 1, Lp), jnp.int32),
            pltpu.VMEM((nbuf, Lp, 128), jnp.float32),
            pltpu.VMEM((Lp,), jnp.float32),
            pltpu.VMEM((Lp,), jnp.float32),
            pltpu.VMEM((_D,), jnp.float32),
            pltpu.VMEM((spw, _D), jnp.float32),
        ] + [pltpu.SemaphoreType.DMA] * nbuf

    @functools.partial(
        pl.kernel,
        out_type=jax.ShapeDtypeStruct((B, _D), jnp.float32),
        mesh=mesh,
        compiler_params=pltpu.CompilerParams(
            needs_layout_passes=False, use_tc_tiling_on_sc=True),
        scratch_types=scratch_types,
    )
    def pool(seqs, table, scale, bias, w, out,
             idx_v, rows_v, scale_v, bias_v, w_v, out_v, *sems):
        wid = lax.axis_index("s") * _NC + lax.axis_index("c")
        base = wid * spw
        pltpu.sync_copy(scale, scale_v)
        pltpu.sync_copy(bias, bias_v)
        pltpu.sync_copy(w, w_v)
        pltpu.sync_copy(seqs.at[pl.ds(base, spw)], idx_v.at[pl.ds(0, spw)])
        for j in range(nbuf - 1):
            for c in range(nch):
                idx_v[spw + j, pl.ds(c * _L16, _L16)] = \
                    wid * 503 + j * 67 + c * _L16 + lax.iota(jnp.int32, _L16)
        wregs = [w_v[pl.ds(q * 16, 16)] for q in range(4)]

        def start(s, b):
            for (o, c) in chunks:
                pltpu.async_copy(table.at[idx_v.at[s, pl.ds(o, c)]],
                                 rows_v.at[b, pl.ds(o, c)], sems[b])

        def wait(b):
            for (o, c) in chunks:
                pltpu.make_async_copy(table.at[idx_v.at[0, pl.ds(o, c)]],
                                      rows_v.at[b, pl.ds(o, c)],
                                      sems[b]).wait()

        def compute(s, b):
            rb = rows_v.at[b]
            lanes = lax.iota(jnp.int32, _L16)

            def chunk_body(c, carry):
                a0, a1, a2, a3, wsv = carry
                terms = []
                for k in range(_L16):
                    l = c * _L16 + k
                    m = rb[l, pl.ds(0, 16)] * wregs[0]
                    for q in range(1, 4):
                        m = m + rb[l, pl.ds(q * 16, 16)] * wregs[q]
                    terms.append(
                        jnp.where(lanes == k, jnp.sum(m), 0.0))
                while len(terms) > 1:
                    terms = [a + b2 for a, b2 in zip(terms[::2], terms[1::2])]
                dots = terms[0]
                ids = idx_v[s, pl.ds(c * _L16, _L16)]
                x = dots * scale_v[pl.ds(c * _L16, _L16)] \
                    + bias_v[pl.ds(c * _L16, _L16)]
                sig = 1.0 / (1.0 + jnp.exp(-x))
                wv16 = jnp.where(ids != 0, sig, 0.0)
                accs = [a0, a1, a2, a3]
                for k in range(_L16):
                    l = c * _L16 + k
                    ws = wv16[k]
                    for q in range(4):
                        accs[q] = accs[q] + ws * rb[l, pl.ds(q * 16, 16)]
                return (accs[0], accs[1], accs[2], accs[3], wsv + wv16)

            z = jnp.zeros((_L16,), jnp.float32)
            a0, a1, a2, a3, wsv = lax.fori_loop(
                0, nch, chunk_body, (z, z, z, z, z))
            r = 1.0 / (jnp.zeros((_L16,), jnp.float32) + jnp.sum(wsv) + _SMALL)
            out_v[s, pl.ds(0, 16)] = a0 * r
            out_v[s, pl.ds(16, 16)] = a1 * r
            out_v[s, pl.ds(32, 16)] = a2 * r
            out_v[s, pl.ds(48, 16)] = a3 * r

        for j in range(nbuf - 1):
            start(j, j)

        def group_body(g, _):
            for b in range(nbuf):
                s = nbuf * g + b
                start(s + nbuf - 1, (b + nbuf - 1) % nbuf)
                wait(b)
                compute(s, b)
            return 0

        lax.fori_loop(0, spw // nbuf, group_body, 0)
        for j in range(nbuf - 1):
            wait(j)
        pltpu.sync_copy(out_v, out.at[pl.ds(base, spw)])

    return pool


def _dot_body(w_ref, t_ref, o_ref):
    o_ref[...] = lax.dot_general(
        w_ref[...], t_ref[...], (((1,), (1,)), ((), ())),
        preferred_element_type=jnp.float32)


def _dot_table(table, W):
    V = table.shape[0]
    BK = 8192
    NB = (V + BK - 1) // BK
    t2d = pl.pallas_call(
        _dot_body,
        grid=(NB,),
        in_specs=[
            pl.BlockSpec((1, _D), lambda i: (0, 0)),
            pl.BlockSpec((BK, _D), lambda i: (i, 0)),
        ],
        out_specs=pl.BlockSpec((1, BK), lambda i: (0, i)),
        out_shape=jax.ShapeDtypeStruct((1, NB * BK), jnp.float32),
    )(W.reshape(1, _D), table)
    return t2d.reshape(-1)[:V]


def _pack_body(t_ref, o_ref):
    o_ref[:, pl.ds(0, _D)] = t_ref[...].T


def _pack_table(table):
    V = table.shape[0]
    BK = 2048
    NB = (V + BK - 1) // BK
    return pl.pallas_call(
        _pack_body,
        grid=(NB,),
        in_specs=[pl.BlockSpec((_D, BK), lambda i: (0, i))],
        out_specs=pl.BlockSpec((BK, 128), lambda i: (i, 0)),
        out_shape=jax.ShapeDtypeStruct((V, 128), jnp.float32),
    )(table.T)


def _sim_body(q_ref, c_ref, o_ref):
    q = q_ref[...]
    c = c_ref[...]
    qn = q / (jnp.sqrt(jnp.sum(q * q, axis=1, keepdims=True)) + _SMALL)
    cn = c / (jnp.sqrt(jnp.sum(c * c, axis=1, keepdims=True)) + _SMALL)
    o_ref[...] = lax.dot_general(
        qn, cn, (((1,), (1,)), ((), ())),
        preferred_element_type=jnp.float32)


def _similarity(qm, cm):
    B = qm.shape[0]
    BQ, BC = 512, 1024
    return pl.pallas_call(
        _sim_body,
        grid=(B // BQ, B // BC),
        in_specs=[
            pl.BlockSpec((BQ, _D), lambda i, j: (i, 0)),
            pl.BlockSpec((BC, _D), lambda i, j: (j, 0)),
        ],
        out_specs=pl.BlockSpec((BQ, BC), lambda i, j: (i, j)),
        out_shape=jax.ShapeDtypeStruct((B, B), jnp.float32),
    )(qm, cm)


def _prep(seqs, gamma, beta, mean, var, L, V):
    Lp = ((L + _L16 - 1) // _L16) * _L16
    B = seqs.shape[0]
    spread = (jax.lax.broadcasted_iota(jnp.int32, (B, Lp), 0) * 131
              + jax.lax.broadcasted_iota(jnp.int32, (B, Lp), 1)) % V
    pos = jax.lax.broadcasted_iota(jnp.int32, (B, Lp), 1)
    seqs_p = jnp.where(pos < L,
                       jnp.pad(seqs.astype(jnp.int32),
                               ((0, 0), (0, Lp - L))),
                       spread)
    sc = gamma / jnp.sqrt(var + _BN_EPS)
    bs = beta - mean * sc
    sc_p = jnp.pad(sc, (0, Lp - L))
    bs_p = jnp.pad(bs, (0, Lp - L), constant_values=-60.0)
    return seqs_p, sc_p, bs_p, Lp


def kernel(code_seqs, query_seqs, code_table, query_table, Wc, Wq,
           gamma_c, beta_c, mean_c, var_c, gamma_q, beta_q, mean_q, var_q):
    B, LC = code_seqs.shape
    _, LQ = query_seqs.shape
    cs, csc, cbs, LCp = _prep(code_seqs, gamma_c, beta_c, mean_c, var_c, LC,
                              code_table.shape[0])
    qs, qsc, qbs, LQp = _prep(query_seqs, gamma_q, beta_q, mean_q, var_q, LQ,
                              query_table.shape[0])
    ct128 = _pack_table(code_table)
    qt128 = _pack_table(query_table)
    cmean = _make_pool(B, LCp, 2)(cs, ct128, csc, cbs, Wc.reshape(-1))
    qmean = _make_pool(B, LQp, 4)(qs, qt128, qsc, qbs, Wq.reshape(-1))
    return _similarity(qmean, cmean)


# speedup vs baseline: 3.2360x; 1.0245x over previous
import functools

import jax
import jax.numpy as jnp
from jax import lax
from jax.experimental import pallas as pl
from jax.experimental.pallas import tpu as pltpu
from jax.experimental.pallas import tpu_sc as plsc

_SMALL = 1e-8
_BN_EPS = 1e-5
_D = 64
_NC = 2
_NS = 16
_NW = _NC * _NS
_L16 = 16


def _make_pool(B, Lp, nbuf):
    nch = Lp // _L16
    spw = B // _NW
    chunks = []
    off = 0
    while off < Lp:
        cl = min(128, Lp - off)
        chunks.append((off, cl))
        off += cl

    mesh = plsc.VectorSubcoreMesh(core_axis_name="c", subcore_axis_name="s")

    scratch_types = [
            pltpu.VMEM((spw + nbuf - 1, Lp), jnp.int32),
            pltpu.VMEM((nbuf, Lp, 128), jnp.float32),
            pltpu.VMEM((Lp,), jnp.float32),
            pltpu.VMEM((Lp,), jnp.float32),
            pltpu.VMEM((spw, _D), jnp.float32),
        ] + [pltpu.SemaphoreType.DMA] * nbuf

    @functools.partial(
        pl.kernel,
        out_type=jax.ShapeDtypeStruct((B, _D), jnp.float32),
        mesh=mesh,
        compiler_params=pltpu.CompilerParams(
            needs_layout_passes=False, use_tc_tiling_on_sc=True),
        scratch_types=scratch_types,
    )
    def pool(seqs, table, scale, bias, out,
             idx_v, rows_v, scale_v, bias_v, out_v, *sems):
        wid = lax.axis_index("s") * _NC + lax.axis_index("c")
        base = wid * spw
        pltpu.sync_copy(scale, scale_v)
        pltpu.sync_copy(bias, bias_v)
        pltpu.sync_copy(seqs.at[pl.ds(base, spw)], idx_v.at[pl.ds(0, spw)])
        for j in range(nbuf - 1):
            for c in range(nch):
                idx_v[spw + j, pl.ds(c * _L16, _L16)] = \
                    wid * 503 + j * 67 + c * _L16 + lax.iota(jnp.int32, _L16)

        def start(s, b):
            for (o, c) in chunks:
                pltpu.async_copy(table.at[idx_v.at[s, pl.ds(o, c)]],
                                 rows_v.at[b, pl.ds(o, c)], sems[b])

        def wait(b):
            for (o, c) in chunks:
                pltpu.make_async_copy(table.at[idx_v.at[0, pl.ds(o, c)]],
                                      rows_v.at[b, pl.ds(o, c)],
                                      sems[b]).wait()

        def compute(s, b):
            rb = rows_v.at[b]
            lanes = lax.iota(jnp.int32, _L16)

            def chunk_body(c, carry):
                a0, a1, a2, a3, wsv = carry
                terms = []
                for k in range(_L16):
                    l = c * _L16 + k
                    terms.append(
                        jnp.where(lanes == k, rb[l, pl.ds(_D, 16)], 0.0))
                while len(terms) > 1:
                    terms = [a + b2 for a, b2 in zip(terms[::2], terms[1::2])]
                dots = terms[0]
                ids = idx_v[s, pl.ds(c * _L16, _L16)]
                x = dots * scale_v[pl.ds(c * _L16, _L16)] \
                    + bias_v[pl.ds(c * _L16, _L16)]
                sig = 1.0 / (1.0 + jnp.exp(-x))
                wv16 = jnp.where(ids != 0, sig, 0.0)
                accs = [a0, a1, a2, a3]
                for k in range(_L16):
                    l = c * _L16 + k
                    ws = wv16[k]
                    for q in range(4):
                        accs[q] = accs[q] + ws * rb[l, pl.ds(q * 16, 16)]
                return (accs[0], accs[1], accs[2], accs[3], wsv + wv16)

            z = jnp.zeros((_L16,), jnp.float32)
            a0, a1, a2, a3, wsv = lax.fori_loop(
                0, nch, chunk_body, (z, z, z, z, z))
            r = 1.0 / (jnp.zeros((_L16,), jnp.float32) + jnp.sum(wsv) + _SMALL)
            out_v[s, pl.ds(0, 16)] = a0 * r
            out_v[s, pl.ds(16, 16)] = a1 * r
            out_v[s, pl.ds(32, 16)] = a2 * r
            out_v[s, pl.ds(48, 16)] = a3 * r

        for j in range(nbuf - 1):
            start(j, j)

        def group_body(g, _):
            for b in range(nbuf):
                s = nbuf * g + b
                start(s + nbuf - 1, (b + nbuf - 1) % nbuf)
                wait(b)
                compute(s, b)
            return 0

        lax.fori_loop(0, spw // nbuf, group_body, 0)
        for j in range(nbuf - 1):
            wait(j)
        pltpu.sync_copy(out_v, out.at[pl.ds(base, spw)])

    return pool


def _dot_body(w_ref, t_ref, o_ref):
    o_ref[...] = lax.dot_general(
        w_ref[...], t_ref[...], (((1,), (1,)), ((), ())),
        preferred_element_type=jnp.float32)


def _dot_table(table, W):
    V = table.shape[0]
    BK = 8192
    NB = (V + BK - 1) // BK
    t2d = pl.pallas_call(
        _dot_body,
        grid=(NB,),
        in_specs=[
            pl.BlockSpec((1, _D), lambda i: (0, 0)),
            pl.BlockSpec((BK, _D), lambda i: (i, 0)),
        ],
        out_specs=pl.BlockSpec((1, BK), lambda i: (0, i)),
        out_shape=jax.ShapeDtypeStruct((1, NB * BK), jnp.float32),
    )(W.reshape(1, _D), table)
    return t2d.reshape(-1)[:V]


def _pack_body(t_ref, m_ref, o_ref):
    o_ref[...] = lax.dot_general(
        t_ref[...], m_ref[...], (((0,), (0,)), ((), ())),
        precision=lax.Precision.HIGHEST,
        preferred_element_type=jnp.float32)


def _pack_table(table, W):
    V = table.shape[0]
    BK = 2048
    NB = (V + BK - 1) // BK
    m = jnp.concatenate(
        [jnp.eye(_D, dtype=jnp.float32),
         jnp.tile(W.reshape(_D, 1), (1, _D))], axis=1)
    return pl.pallas_call(
        _pack_body,
        grid=(NB,),
        in_specs=[
            pl.BlockSpec((_D, BK), lambda i: (0, i)),
            pl.BlockSpec((_D, 128), lambda i: (0, 0)),
        ],
        out_specs=pl.BlockSpec((BK, 128), lambda i: (i, 0)),
        out_shape=jax.ShapeDtypeStruct((V, 128), jnp.float32),
    )(table.T, m)


def _sim_body(q_ref, c_ref, o_ref):
    q = q_ref[...]
    c = c_ref[...]
    qn = q / (jnp.sqrt(jnp.sum(q * q, axis=1, keepdims=True)) + _SMALL)
    cn = c / (jnp.sqrt(jnp.sum(c * c, axis=1, keepdims=True)) + _SMALL)
    o_ref[...] = lax.dot_general(
        qn, cn, (((1,), (1,)), ((), ())),
        preferred_element_type=jnp.float32)


def _similarity(qm, cm):
    B = qm.shape[0]
    BQ, BC = 512, 1024
    return pl.pallas_call(
        _sim_body,
        grid=(B // BQ, B // BC),
        in_specs=[
            pl.BlockSpec((BQ, _D), lambda i, j: (i, 0)),
            pl.BlockSpec((BC, _D), lambda i, j: (j, 0)),
        ],
        out_specs=pl.BlockSpec((BQ, BC), lambda i, j: (i, j)),
        out_shape=jax.ShapeDtypeStruct((B, B), jnp.float32),
    )(qm, cm)


def _prep(seqs, gamma, beta, mean, var, L, V):
    Lp = ((L + _L16 - 1) // _L16) * _L16
    B = seqs.shape[0]
    spread = (jax.lax.broadcasted_iota(jnp.int32, (B, Lp), 0) * 131
              + jax.lax.broadcasted_iota(jnp.int32, (B, Lp), 1)) % V
    pos = jax.lax.broadcasted_iota(jnp.int32, (B, Lp), 1)
    seqs_p = jnp.where(pos < L,
                       jnp.pad(seqs.astype(jnp.int32),
                               ((0, 0), (0, Lp - L))),
                       spread)
    sc = gamma / jnp.sqrt(var + _BN_EPS)
    bs = beta - mean * sc
    sc_p = jnp.pad(sc, (0, Lp - L))
    bs_p = jnp.pad(bs, (0, Lp - L), constant_values=-60.0)
    return seqs_p, sc_p, bs_p, Lp


def kernel(code_seqs, query_seqs, code_table, query_table, Wc, Wq,
           gamma_c, beta_c, mean_c, var_c, gamma_q, beta_q, mean_q, var_q):
    B, LC = code_seqs.shape
    _, LQ = query_seqs.shape
    cs, csc, cbs, LCp = _prep(code_seqs, gamma_c, beta_c, mean_c, var_c, LC,
                              code_table.shape[0])
    qs, qsc, qbs, LQp = _prep(query_seqs, gamma_q, beta_q, mean_q, var_q, LQ,
                              query_table.shape[0])
    ct128 = _pack_table(code_table, Wc)
    qt128 = _pack_table(query_table, Wq)
    cmean = _make_pool(B, LCp, 2)(cs, ct128, csc, cbs)
    qmean = _make_pool(B, LQp, 4)(qs, qt128, qsc, qbs)
    return _similarity(qmean, cmean)


# speedup vs baseline: 3.5119x; 1.0853x over previous
import functools

import jax
import jax.numpy as jnp
from jax import lax
from jax.experimental import pallas as pl
from jax.experimental.pallas import tpu as pltpu
from jax.experimental.pallas import tpu_sc as plsc

_SMALL = 1e-8
_BN_EPS = 1e-5
_D = 64
_NC = 2
_NS = 16
_NW = _NC * _NS
_L16 = 16


def _make_pool(B, Lp, nbuf):
    nch = Lp // _L16
    spw = B // _NW
    chunks = []
    off = 0
    while off < Lp:
        cl = min(128, Lp - off)
        chunks.append((off, cl))
        off += cl

    mesh = plsc.VectorSubcoreMesh(core_axis_name="c", subcore_axis_name="s")

    scratch_types = [
            pltpu.VMEM((spw + nbuf - 1, Lp), jnp.int32),
            pltpu.VMEM((nbuf, Lp, 128), jnp.float32),
            pltpu.VMEM((Lp,), jnp.float32),
            pltpu.VMEM((Lp,), jnp.float32),
            pltpu.VMEM((spw, _D), jnp.float32),
        ] + [pltpu.SemaphoreType.DMA] * nbuf

    @functools.partial(
        pl.kernel,
        out_type=jax.ShapeDtypeStruct((B, _D), jnp.float32),
        mesh=mesh,
        compiler_params=pltpu.CompilerParams(
            needs_layout_passes=False, use_tc_tiling_on_sc=True),
        scratch_types=scratch_types,
    )
    def pool(seqs, table, scale, bias, out,
             idx_v, rows_v, scale_v, bias_v, out_v, *sems):
        wid = lax.axis_index("s") * _NC + lax.axis_index("c")
        base = wid * spw
        pltpu.sync_copy(scale, scale_v)
        pltpu.sync_copy(bias, bias_v)
        pltpu.sync_copy(seqs.at[pl.ds(base, spw)], idx_v.at[pl.ds(0, spw)])
        for j in range(nbuf - 1):
            for c in range(nch):
                idx_v[spw + j, pl.ds(c * _L16, _L16)] = \
                    wid * 503 + j * 67 + c * _L16 + lax.iota(jnp.int32, _L16)

        def start(s, b):
            for (o, c) in chunks:
                pltpu.async_copy(table.at[idx_v.at[s, pl.ds(o, c)]],
                                 rows_v.at[b, pl.ds(o, c)], sems[b])

        def wait(b):
            for (o, c) in chunks:
                pltpu.make_async_copy(table.at[idx_v.at[0, pl.ds(o, c)]],
                                      rows_v.at[b, pl.ds(o, c)],
                                      sems[b]).wait()

        def compute(s, b):
            rb = rows_v.at[b]
            lanes = lax.iota(jnp.int32, _L16)

            def chunk_body(c, carry):
                a0, a1, a2, a3, wsv = carry
                terms = []
                for k in range(_L16):
                    l = c * _L16 + k
                    terms.append(
                        jnp.where(lanes == k, rb[l, pl.ds(_D, 16)], 0.0))
                while len(terms) > 1:
                    terms = [a + b2 for a, b2 in zip(terms[::2], terms[1::2])]
                dots = terms[0]
                ids = idx_v[s, pl.ds(c * _L16, _L16)]
                x = dots * scale_v[pl.ds(c * _L16, _L16)] \
                    + bias_v[pl.ds(c * _L16, _L16)]
                sig = 1.0 / (1.0 + jnp.exp(-x))
                wv16 = jnp.where(ids != 0, sig, 0.0)
                accs = [a0, a1, a2, a3]
                for k in range(_L16):
                    l = c * _L16 + k
                    ws = wv16[k]
                    for q in range(4):
                        accs[q] = accs[q] + ws * rb[l, pl.ds(q * 16, 16)]
                return (accs[0], accs[1], accs[2], accs[3], wsv + wv16)

            z = jnp.zeros((_L16,), jnp.float32)
            a0, a1, a2, a3, wsv = lax.fori_loop(
                0, nch, chunk_body, (z, z, z, z, z))
            r = 1.0 / (jnp.zeros((_L16,), jnp.float32) + jnp.sum(wsv) + _SMALL)
            out_v[s, pl.ds(0, 16)] = a0 * r
            out_v[s, pl.ds(16, 16)] = a1 * r
            out_v[s, pl.ds(32, 16)] = a2 * r
            out_v[s, pl.ds(48, 16)] = a3 * r

        for j in range(nbuf - 1):
            start(j, j)

        def group_body(g, _):
            for b in range(nbuf):
                s = nbuf * g + b
                start(s + nbuf - 1, (b + nbuf - 1) % nbuf)
                wait(b)
                compute(s, b)
            return 0

        lax.fori_loop(0, spw // nbuf, group_body, 0)
        for j in range(nbuf - 1):
            wait(j)
        pltpu.sync_copy(out_v, out.at[pl.ds(base, spw)])

    return pool


def _dot_body(w_ref, t_ref, o_ref):
    o_ref[...] = lax.dot_general(
        w_ref[...], t_ref[...], (((1,), (1,)), ((), ())),
        preferred_element_type=jnp.float32)


def _dot_table(table, W):
    V = table.shape[0]
    BK = 8192
    NB = (V + BK - 1) // BK
    t2d = pl.pallas_call(
        _dot_body,
        grid=(NB,),
        in_specs=[
            pl.BlockSpec((1, _D), lambda i: (0, 0)),
            pl.BlockSpec((BK, _D), lambda i: (i, 0)),
        ],
        out_specs=pl.BlockSpec((1, BK), lambda i: (0, i)),
        out_shape=jax.ShapeDtypeStruct((1, NB * BK), jnp.float32),
    )(W.reshape(1, _D), table)
    return t2d.reshape(-1)[:V]


def _pack_body(t_ref, m_ref, o_ref):
    x = t_ref[...]
    o_ref[:, pl.ds(0, _D)] = lax.dot_general(
        x, m_ref[:, pl.ds(0, _D)], (((0,), (0,)), ((), ())),
        precision=lax.Precision.HIGHEST,
        preferred_element_type=jnp.float32)
    o_ref[:, pl.ds(_D, _D)] = lax.dot_general(
        x, m_ref[:, pl.ds(_D, _D)], (((0,), (0,)), ((), ())),
        preferred_element_type=jnp.float32)


def _pack_table(table, W):
    V = table.shape[0]
    BK = 4096
    NB = (V + BK - 1) // BK
    m = jnp.concatenate(
        [jnp.eye(_D, dtype=jnp.float32),
         jnp.tile(W.reshape(_D, 1), (1, _D))], axis=1)
    return pl.pallas_call(
        _pack_body,
        grid=(NB,),
        in_specs=[
            pl.BlockSpec((_D, BK), lambda i: (0, i)),
            pl.BlockSpec((_D, 128), lambda i: (0, 0)),
        ],
        out_specs=pl.BlockSpec((BK, 128), lambda i: (i, 0)),
        out_shape=jax.ShapeDtypeStruct((V, 128), jnp.float32),
    )(table.T, m)


def _sim_body(q_ref, c_ref, o_ref):
    q = q_ref[...]
    c = c_ref[...]
    qn = q / (jnp.sqrt(jnp.sum(q * q, axis=1, keepdims=True)) + _SMALL)
    cn = c / (jnp.sqrt(jnp.sum(c * c, axis=1, keepdims=True)) + _SMALL)
    o_ref[...] = lax.dot_general(
        qn, cn, (((1,), (1,)), ((), ())),
        preferred_element_type=jnp.float32)


def _similarity(qm, cm):
    B = qm.shape[0]
    BQ, BC = 512, 1024
    return pl.pallas_call(
        _sim_body,
        grid=(B // BQ, B // BC),
        in_specs=[
            pl.BlockSpec((BQ, _D), lambda i, j: (i, 0)),
            pl.BlockSpec((BC, _D), lambda i, j: (j, 0)),
        ],
        out_specs=pl.BlockSpec((BQ, BC), lambda i, j: (i, j)),
        out_shape=jax.ShapeDtypeStruct((B, B), jnp.float32),
    )(qm, cm)


def _prep(seqs, gamma, beta, mean, var, L, V):
    Lp = ((L + _L16 - 1) // _L16) * _L16
    B = seqs.shape[0]
    spread = (jax.lax.broadcasted_iota(jnp.int32, (B, Lp), 0) * 131
              + jax.lax.broadcasted_iota(jnp.int32, (B, Lp), 1)) % V
    pos = jax.lax.broadcasted_iota(jnp.int32, (B, Lp), 1)
    seqs_p = jnp.where(pos < L,
                       jnp.pad(seqs.astype(jnp.int32),
                               ((0, 0), (0, Lp - L))),
                       spread)
    sc = gamma / jnp.sqrt(var + _BN_EPS)
    bs = beta - mean * sc
    sc_p = jnp.pad(sc, (0, Lp - L))
    bs_p = jnp.pad(bs, (0, Lp - L), constant_values=-60.0)
    return seqs_p, sc_p, bs_p, Lp


def kernel(code_seqs, query_seqs, code_table, query_table, Wc, Wq,
           gamma_c, beta_c, mean_c, var_c, gamma_q, beta_q, mean_q, var_q):
    B, LC = code_seqs.shape
    _, LQ = query_seqs.shape
    cs, csc, cbs, LCp = _prep(code_seqs, gamma_c, beta_c, mean_c, var_c, LC,
                              code_table.shape[0])
    qs, qsc, qbs, LQp = _prep(query_seqs, gamma_q, beta_q, mean_q, var_q, LQ,
                              query_table.shape[0])
    ct128 = _pack_table(code_table, Wc)
    qt128 = _pack_table(query_table, Wq)
    cmean = _make_pool(B, LCp, 2)(cs, ct128, csc, cbs)
    qmean = _make_pool(B, LQp, 4)(qs, qt128, qsc, qbs)
    return _similarity(qmean, cmean)


# speedup vs baseline: 3.5144x; 1.0007x over previous
import functools

import jax
import jax.numpy as jnp
from jax import lax
from jax.experimental import pallas as pl
from jax.experimental.pallas import tpu as pltpu
from jax.experimental.pallas import tpu_sc as plsc

_SMALL = 1e-8
_BN_EPS = 1e-5
_D = 64
_NC = 2
_NS = 16
_NW = _NC * _NS
_L16 = 16


def _make_pool(B, Lp, nbuf):
    nch = Lp // _L16
    spw = B // _NW
    chunks = []
    off = 0
    while off < Lp:
        cl = min(128, Lp - off)
        chunks.append((off, cl))
        off += cl

    mesh = plsc.VectorSubcoreMesh(core_axis_name="c", subcore_axis_name="s")

    scratch_types = [
            pltpu.VMEM((spw + nbuf - 1, Lp), jnp.int32),
            pltpu.VMEM((nbuf, Lp, 128), jnp.float32),
            pltpu.VMEM((Lp,), jnp.float32),
            pltpu.VMEM((Lp,), jnp.float32),
            pltpu.VMEM((spw, _D), jnp.float32),
        ] + [pltpu.SemaphoreType.DMA] * nbuf

    @functools.partial(
        pl.kernel,
        out_type=jax.ShapeDtypeStruct((B, _D), jnp.float32),
        mesh=mesh,
        compiler_params=pltpu.CompilerParams(
            needs_layout_passes=False, use_tc_tiling_on_sc=True),
        scratch_types=scratch_types,
    )
    def pool(seqs, table, scale, bias, out,
             idx_v, rows_v, scale_v, bias_v, out_v, *sems):
        wid = lax.axis_index("s") * _NC + lax.axis_index("c")
        base = wid * spw
        pltpu.sync_copy(scale, scale_v)
        pltpu.sync_copy(bias, bias_v)
        pltpu.sync_copy(seqs.at[pl.ds(base, spw)], idx_v.at[pl.ds(0, spw)])
        for j in range(nbuf - 1):
            for c in range(nch):
                idx_v[spw + j, pl.ds(c * _L16, _L16)] = \
                    wid * 503 + j * 67 + c * _L16 + lax.iota(jnp.int32, _L16)

        def start(s, b):
            for (o, c) in chunks:
                pltpu.async_copy(table.at[idx_v.at[s, pl.ds(o, c)]],
                                 rows_v.at[b, pl.ds(o, c)], sems[b])

        def wait(b):
            for (o, c) in chunks:
                pltpu.make_async_copy(table.at[idx_v.at[0, pl.ds(o, c)]],
                                      rows_v.at[b, pl.ds(o, c)],
                                      sems[b]).wait()

        def compute(s, b):
            rb = rows_v.at[b]
            lanes = lax.iota(jnp.int32, _L16)

            def chunk_body(c, carry):
                a0, a1, a2, a3, wsv = carry
                terms = []
                for k in range(_L16):
                    l = c * _L16 + k
                    terms.append(
                        jnp.where(lanes == k, rb[l, pl.ds(_D, 16)], 0.0))
                while len(terms) > 1:
                    terms = [a + b2 for a, b2 in zip(terms[::2], terms[1::2])]
                dots = terms[0]
                ids = idx_v[s, pl.ds(c * _L16, _L16)]
                x = dots * scale_v[pl.ds(c * _L16, _L16)] \
                    + bias_v[pl.ds(c * _L16, _L16)]
                sig = 1.0 / (1.0 + jnp.exp(-x))
                wv16 = jnp.where(ids != 0, sig, 0.0)
                accs = [a0, a1, a2, a3]
                for k in range(_L16):
                    l = c * _L16 + k
                    ws = wv16[k]
                    for q in range(4):
                        accs[q] = accs[q] + ws * rb[l, pl.ds(q * 16, 16)]
                return (accs[0], accs[1], accs[2], accs[3], wsv + wv16)

            z = jnp.zeros((_L16,), jnp.float32)
            a0, a1, a2, a3, wsv = lax.fori_loop(
                0, nch, chunk_body, (z, z, z, z, z))
            r = 1.0 / (jnp.zeros((_L16,), jnp.float32) + jnp.sum(wsv) + _SMALL)
            out_v[s, pl.ds(0, 16)] = a0 * r
            out_v[s, pl.ds(16, 16)] = a1 * r
            out_v[s, pl.ds(32, 16)] = a2 * r
            out_v[s, pl.ds(48, 16)] = a3 * r

        for j in range(nbuf - 1):
            start(j, j)

        def group_body(g, _):
            for b in range(nbuf):
                s = nbuf * g + b
                start(s + nbuf - 1, (b + nbuf - 1) % nbuf)
                wait(b)
                compute(s, b)
            return 0

        lax.fori_loop(0, spw // nbuf, group_body, 0)
        for j in range(nbuf - 1):
            wait(j)
        pltpu.sync_copy(out_v, out.at[pl.ds(base, spw)])

    return pool


def _pack_body(t_ref, m_ref, o_ref):
    x = t_ref[...]
    o_ref[:, pl.ds(0, _D)] = lax.dot_general(
        x, m_ref[:, pl.ds(0, _D)], (((0,), (0,)), ((), ())),
        precision=lax.Precision.HIGHEST,
        preferred_element_type=jnp.float32)
    o_ref[:, pl.ds(_D, _D)] = lax.dot_general(
        x, m_ref[:, pl.ds(_D, _D)], (((0,), (0,)), ((), ())),
        preferred_element_type=jnp.float32)


def _pack_table(table, W):
    V = table.shape[0]
    BK = 4096
    NB = (V + BK - 1) // BK
    m = jnp.concatenate(
        [jnp.eye(_D, dtype=jnp.float32),
         jnp.tile(W.reshape(_D, 1), (1, _D))], axis=1)
    return pl.pallas_call(
        _pack_body,
        grid=(NB,),
        in_specs=[
            pl.BlockSpec((_D, BK), lambda i: (0, i)),
            pl.BlockSpec((_D, 128), lambda i: (0, 0)),
        ],
        out_specs=pl.BlockSpec((BK, 128), lambda i: (i, 0)),
        out_shape=jax.ShapeDtypeStruct((V, 128), jnp.float32),
    )(table.T, m)


def _sim_body(q_ref, c_ref, o_ref):
    q = q_ref[...]
    c = c_ref[...]
    qn = q / (jnp.sqrt(jnp.sum(q * q, axis=1, keepdims=True)) + _SMALL)
    cn = c / (jnp.sqrt(jnp.sum(c * c, axis=1, keepdims=True)) + _SMALL)
    o_ref[...] = lax.dot_general(
        qn, cn, (((1,), (1,)), ((), ())),
        preferred_element_type=jnp.float32)


def _similarity(qm, cm):
    B = qm.shape[0]
    BQ, BC = 512, 1024
    return pl.pallas_call(
        _sim_body,
        grid=(B // BQ, B // BC),
        in_specs=[
            pl.BlockSpec((BQ, _D), lambda i, j: (i, 0)),
            pl.BlockSpec((BC, _D), lambda i, j: (j, 0)),
        ],
        out_specs=pl.BlockSpec((BQ, BC), lambda i, j: (i, j)),
        out_shape=jax.ShapeDtypeStruct((B, B), jnp.float32),
    )(qm, cm)


def _prep(seqs, gamma, beta, mean, var, L, V):
    Lp = ((L + _L16 - 1) // _L16) * _L16
    B = seqs.shape[0]
    spread = (jax.lax.broadcasted_iota(jnp.int32, (B, Lp), 0) * 131
              + jax.lax.broadcasted_iota(jnp.int32, (B, Lp), 1)) % V
    pos = jax.lax.broadcasted_iota(jnp.int32, (B, Lp), 1)
    seqs_p = jnp.where(pos < L,
                       jnp.pad(seqs.astype(jnp.int32),
                               ((0, 0), (0, Lp - L))),
                       spread)
    sc = gamma / jnp.sqrt(var + _BN_EPS)
    bs = beta - mean * sc
    sc_p = jnp.pad(sc, (0, Lp - L))
    bs_p = jnp.pad(bs, (0, Lp - L), constant_values=-60.0)
    return seqs_p, sc_p, bs_p, Lp


def kernel(code_seqs, query_seqs, code_table, query_table, Wc, Wq,
           gamma_c, beta_c, mean_c, var_c, gamma_q, beta_q, mean_q, var_q):
    B, LC = code_seqs.shape
    _, LQ = query_seqs.shape
    cs, csc, cbs, LCp = _prep(code_seqs, gamma_c, beta_c, mean_c, var_c, LC,
                              code_table.shape[0])
    qs, qsc, qbs, LQp = _prep(query_seqs, gamma_q, beta_q, mean_q, var_q, LQ,
                              query_table.shape[0])
    ct128 = _pack_table(code_table, Wc)
    qt128 = _pack_table(query_table, Wq)
    cmean = _make_pool(B, LCp, 2)(cs, ct128, csc, cbs)
    qmean = _make_pool(B, LQp, 4)(qs, qt128, qsc, qbs)
    return _similarity(qmean, cmean)
